# baseline (speedup 1.0000x reference)
import functools

import jax
import jax.numpy as jnp
from jax import lax
from jax.experimental import pallas as pl
from jax.experimental.pallas import tpu as pltpu
from jax.experimental.pallas import tpu_sc as plsc

N_ROWS = 8192
DIM = 4096
HDIM = DIM // 2
NC = 2
NS = 16
L = 16
NW = NC * NS
ROWS_PER_W = N_ROWS // NW
RBLK = 8
NB = ROWS_PER_W // RBLK

_mesh = plsc.VectorSubcoreMesh(core_axis_name="c", subcore_axis_name="s")


@functools.partial(
    pl.kernel,
    mesh=_mesh,
    out_type=jax.ShapeDtypeStruct((N_ROWS, DIM), jnp.float32),
    compiler_params=pltpu.CompilerParams(needs_layout_passes=False),
    scratch_types=[
        pltpu.VMEM((DIM,), jnp.int32),
        pltpu.VMEM((RBLK, DIM), jnp.float32),
        pltpu.VMEM((RBLK, DIM), jnp.float32),
        pltpu.VMEM((RBLK, HDIM), jnp.float32),
        pltpu.VMEM((RBLK, HDIM), jnp.float32),
        pltpu.SemaphoreType.DMA,
        pltpu.SemaphoreType.DMA,
        pltpu.SemaphoreType.DMA,
        pltpu.SemaphoreType.DMA,
    ],
)
def _permute_cols(x_hbm, perm_hbm, out_hbm, perm_v,
                  in0, in1, out0, out1,
                  sem_in0, sem_in1, sem_out0, sem_out1):
    wid = lax.axis_index("s") * NC + lax.axis_index("c")
    base = wid * ROWS_PER_W
    pltpu.sync_copy(perm_hbm, perm_v)

    def src(b):
        return x_hbm.at[pl.ds(base + b * RBLK, RBLK)]

    def dsth(b, h):
        return out_hbm.at[pl.ds(base + b * RBLK, RBLK), pl.ds(h * HDIM, HDIM)]

    def compute(in_v, out_v, colbase):
        @plsc.parallel_loop(0, HDIM // L, unroll=8)
        def j_body(j):
            col0 = j * L
            idx = perm_v[pl.ds(colbase + col0, L)]
            for r in range(RBLK):
                row_idx = jnp.full((L,), r, jnp.int32)
                out_v[r, pl.ds(col0, L)] = plsc.load_gather(
                    in_v, [row_idx, idx])

    pltpu.async_copy(src(0), in0, sem_in0)

    def block(b, in_v, sem_in, prefetch_in, prefetch_sem):
        @pl.when(b + 1 < NB)
        def _():
            pltpu.async_copy(src(b + 1), prefetch_in, prefetch_sem)

        pltpu.make_async_copy(src(b), in_v, sem_in).wait()

        @pl.when(b > 0)
        def _():
            pltpu.make_async_copy(out0, dsth(b - 1, 0), sem_out0).wait()

        compute(in_v, out0, 0)
        pltpu.async_copy(out0, dsth(b, 0), sem_out0)

        @pl.when(b > 0)
        def _():
            pltpu.make_async_copy(out1, dsth(b - 1, 1), sem_out1).wait()

        compute(in_v, out1, HDIM)
        pltpu.async_copy(out1, dsth(b, 1), sem_out1)

    def pair_body(p, carry):
        b = 2 * p
        block(b, in0, sem_in0, in1, sem_in1)
        block(b + 1, in1, sem_in1, in0, sem_in0)
        return carry

    lax.fori_loop(0, NB // 2, pair_body, 0)
    pltpu.make_async_copy(out0, dsth(NB - 1, 0), sem_out0).wait()
    pltpu.make_async_copy(out1, dsth(NB - 1, 1), sem_out1).wait()


def kernel(x, perm):
    out = _permute_cols(x, perm.astype(jnp.int32))
    return out, 0


# speedup vs baseline: 1.0007x; 1.0007x over previous
import functools

import jax
import jax.numpy as jnp
from jax import lax
from jax.experimental import pallas as pl
from jax.experimental.pallas import tpu as pltpu
from jax.experimental.pallas import tpu_sc as plsc

N_ROWS = 8192
DIM = 4096
HDIM = DIM // 2
NC = 2
NS = 16
L = 16
NW = NC * NS
ROWS_PER_W = N_ROWS // NW
RBLK = 8
NB = ROWS_PER_W // RBLK

_mesh = plsc.VectorSubcoreMesh(core_axis_name="c", subcore_axis_name="s")


@functools.partial(
    pl.kernel,
    mesh=_mesh,
    out_type=jax.ShapeDtypeStruct((N_ROWS, DIM), jnp.float32),
    compiler_params=pltpu.CompilerParams(needs_layout_passes=False),
    scratch_types=[
        pltpu.VMEM((DIM,), jnp.int32),
        pltpu.VMEM((RBLK, DIM), jnp.float32),
        pltpu.VMEM((RBLK, DIM), jnp.float32),
        pltpu.VMEM((RBLK, HDIM), jnp.float32),
        pltpu.VMEM((RBLK, HDIM), jnp.float32),
        pltpu.SemaphoreType.DMA,
        pltpu.SemaphoreType.DMA,
        pltpu.SemaphoreType.DMA,
        pltpu.SemaphoreType.DMA,
    ],
)
def _permute_cols(x_hbm, perm_hbm, out_hbm, perm_v,
                  in0, in1, out0, out1,
                  sem_in0, sem_in1, sem_out0, sem_out1):
    wid = lax.axis_index("s") * NC + lax.axis_index("c")
    base = wid * ROWS_PER_W
    pltpu.sync_copy(perm_hbm, perm_v)

    def src(b):
        return x_hbm.at[pl.ds(base + b * RBLK, RBLK)]

    def dsth(b, h):
        return out_hbm.at[pl.ds(base + b * RBLK, RBLK), pl.ds(h * HDIM, HDIM)]

    def compute(in_v, out_v, colbase):
        @plsc.parallel_loop(0, HDIM // L, unroll=4)
        def j_body(j):
            col0 = j * L
            idx = perm_v[pl.ds(colbase + col0, L)]
            for r in range(RBLK):
                row_idx = jnp.full((L,), r, jnp.int32)
                out_v[r, pl.ds(col0, L)] = plsc.load_gather(
                    in_v, [row_idx, idx])

    pltpu.async_copy(src(0), in0, sem_in0)

    def block(b, in_v, sem_in, prefetch_in, prefetch_sem):
        @pl.when(b + 1 < NB)
        def _():
            pltpu.async_copy(src(b + 1), prefetch_in, prefetch_sem)

        pltpu.make_async_copy(src(b), in_v, sem_in).wait()

        @pl.when(b > 0)
        def _():
            pltpu.make_async_copy(out0, dsth(b - 1, 0), sem_out0).wait()

        compute(in_v, out0, 0)
        pltpu.async_copy(out0, dsth(b, 0), sem_out0)

        @pl.when(b > 0)
        def _():
            pltpu.make_async_copy(out1, dsth(b - 1, 1), sem_out1).wait()

        compute(in_v, out1, HDIM)
        pltpu.async_copy(out1, dsth(b, 1), sem_out1)

    def pair_body(p, carry):
        b = 2 * p
        block(b, in0, sem_in0, in1, sem_in1)
        block(b + 1, in1, sem_in1, in0, sem_in0)
        return carry

    lax.fori_loop(0, NB // 2, pair_body, 0)
    pltpu.make_async_copy(out0, dsth(NB - 1, 0), sem_out0).wait()
    pltpu.make_async_copy(out1, dsth(NB - 1, 1), sem_out1).wait()


def kernel(x, perm):
    out = _permute_cols(x, perm.astype(jnp.int32))
    return out, 0
